# baseline (speedup 1.0000x reference)
import functools

import jax
import jax.numpy as jnp
from jax import lax
from jax.experimental import pallas as pl
from jax.experimental.pallas import tpu as pltpu
from jax.experimental.pallas import tpu_sc as plsc

_TILE_V = 2048
_NEG = -3e38
_LOG2E = 1.4426950408889634
_LN2 = 0.6931471805599453


@functools.lru_cache(maxsize=None)
def _make_sc_gather(B, Wn, V, D):
    info = plsc.get_sparse_core_info()
    NC, NS = info.num_cores, info.num_subcores
    NW = NC * NS
    assert B % NW == 0
    b_per_w = B // NW
    c_per_w = (B * Wn) // NW
    CH = 128
    assert c_per_w % CH == 0 and CH % Wn == 0
    n_ch = c_per_w // CH
    bpc = CH // Wn
    assert D % 16 == 0
    nl = D // 16

    mesh = plsc.VectorSubcoreMesh(core_axis_name="c", subcore_axis_name="s")

    scratch = [pltpu.VMEM((b_per_w,), jnp.int32),
               pltpu.VMEM((b_per_w, D), jnp.float32)]
    scratch += [pltpu.VMEM((CH,), jnp.int32) for _ in range(n_ch)]
    scratch += [pltpu.VMEM((CH, D), jnp.float32) for _ in range(n_ch)]
    scratch += [pltpu.VMEM((b_per_w, D), jnp.float32),
                pltpu.SemaphoreType.DMA]

    @functools.partial(
        pl.kernel,
        mesh=mesh,
        out_type=(jax.ShapeDtypeStruct((B, D), jnp.float32),
                  jax.ShapeDtypeStruct((B, D), jnp.float32)),
        scratch_types=scratch,
        compiler_params=pltpu.CompilerParams(use_tc_tiling_on_sc=False),
    )
    def sc_gather(cidx_hbm, ctx_hbm, wemb_hbm, wout_hbm, out_h, out_csum,
                  idx_h, rows_h, *rest):
        idx_c = rest[0:n_ch]
        rows_c = rest[n_ch:2 * n_ch]
        csum_v = rest[2 * n_ch]
        sem = rest[2 * n_ch + 1]

        wid = lax.axis_index("s") * NC + lax.axis_index("c")
        hbase = wid * b_per_w
        cbase = wid * c_per_w

        pltpu.sync_copy(cidx_hbm.at[pl.ds(hbase, b_per_w)], idx_h)
        for ch in range(n_ch):
            pltpu.sync_copy(ctx_hbm.at[pl.ds(cbase + ch * CH, CH)], idx_c[ch])

        cps = [pltpu.async_copy(wemb_hbm.at[idx_h], rows_h, sem)]
        for ch in range(n_ch):
            cps.append(pltpu.async_copy(wout_hbm.at[idx_c[ch]], rows_c[ch], sem))
        for cp in cps:
            cp.wait()

        for ch in range(n_ch):
            rc = rows_c[ch]

            def wbody(i, carry, rc=rc, off=ch * bpc):
                for l in range(nl):
                    acc = rc[i * Wn, pl.ds(l * 16, 16)]
                    for w in range(1, Wn):
                        acc = acc + rc[i * Wn + w, pl.ds(l * 16, 16)]
                    csum_v[off + i, pl.ds(l * 16, 16)] = acc
                return carry

            lax.fori_loop(0, bpc, wbody, 0)

        pltpu.sync_copy(rows_h, out_h.at[pl.ds(hbase, b_per_w)])
        pltpu.sync_copy(csum_v, out_csum.at[pl.ds(hbase, b_per_w)])

    return sc_gather


def _tc_body(h_ref, csum_ref, wout_ref, out_ref, m_ref, s_ref, *, B, Wn, V,
             tile_v):
    i = pl.program_id(0)
    nt = pl.num_programs(0)

    @pl.when(i == 0)
    def _init():
        m_ref[...] = jnp.full(m_ref.shape, _NEG, jnp.float32)
        s_ref[...] = jnp.zeros(s_ref.shape, jnp.float32)

    hb = (h_ref[...] * _LOG2E).astype(jnp.bfloat16)
    wb = wout_ref[...].astype(jnp.bfloat16)
    l2 = lax.dot_general(hb, wb, (((1,), (1,)), ((), ())),
                         preferred_element_type=jnp.float32)
    l2 = lax.cond(
        i == nt - 1,
        lambda x: jnp.where(
            i * tile_v + lax.broadcasted_iota(jnp.int32, (1, tile_v), 1) < V,
            x, _NEG),
        lambda x: x,
        l2)

    m_old = m_ref[...]
    m_new = jnp.maximum(m_old, jnp.max(l2, axis=1, keepdims=True))
    s_ref[...] = (s_ref[...] * jnp.exp2(m_old - m_new)
                  + jnp.sum(jnp.exp2(l2 - m_new), axis=1, keepdims=True))
    m_ref[...] = m_new

    @pl.when(i == nt - 1)
    def _fin():
        lse = m_ref[...] * _LN2 + jnp.log(s_ref[...])
        ctx_total = jnp.sum(h_ref[...] * csum_ref[...])
        out_ref[0, 0] = jnp.sum(lse) / B - ctx_total / (B * Wn)


@functools.lru_cache(maxsize=None)
def _make_tc_loss(B, Wn, V, D):
    tile_v = _TILE_V
    nt = pl.cdiv(V, tile_v)
    body = functools.partial(_tc_body, B=B, Wn=Wn, V=V, tile_v=tile_v)
    return pl.pallas_call(
        body,
        grid=(nt,),
        in_specs=[
            pl.BlockSpec((B, D), lambda i: (0, 0)),
            pl.BlockSpec((B, D), lambda i: (0, 0)),
            pl.BlockSpec((tile_v, D), lambda i: (i, 0)),
        ],
        out_specs=pl.BlockSpec((1, 1), lambda i: (0, 0),
                               memory_space=pltpu.SMEM),
        out_shape=jax.ShapeDtypeStruct((1, 1), jnp.float32),
        scratch_shapes=[
            pltpu.VMEM((B, 1), jnp.float32),
            pltpu.VMEM((B, 1), jnp.float32),
        ],
    )


def kernel(center_index, context_indices, W_emb, W_out):
    B, Wn = context_indices.shape
    V, D = W_emb.shape
    cidx = center_index.astype(jnp.int32)
    ctx = context_indices.astype(jnp.int32).reshape(-1)
    h, csum = _make_sc_gather(B, Wn, V, D)(cidx, ctx, W_emb, W_out)
    loss = _make_tc_loss(B, Wn, V, D)(h, csum, W_out)
    return loss[0, 0]


# speedup vs baseline: 1.6896x; 1.6896x over previous
import functools

import jax
import jax.numpy as jnp
from jax import lax
from jax.experimental import pallas as pl
from jax.experimental.pallas import tpu as pltpu
from jax.experimental.pallas import tpu_sc as plsc

_TILE_V = 2000
_NEG = -3e38
_LOG2E = 1.4426950408889634
_LN2 = 0.6931471805599453


@functools.lru_cache(maxsize=None)
def _make_sc_gather(B, Wn, V, D):
    info = plsc.get_sparse_core_info()
    NC, NS = info.num_cores, info.num_subcores
    NW = NC * NS
    assert B % NW == 0
    b_per_w = B // NW
    c_per_w = (B * Wn) // NW
    CH = 128
    assert c_per_w % CH == 0 and CH % Wn == 0
    n_ch = c_per_w // CH
    bpc = CH // Wn
    assert D % 16 == 0
    nl = D // 16

    mesh = plsc.VectorSubcoreMesh(core_axis_name="c", subcore_axis_name="s")

    scratch = [pltpu.VMEM((b_per_w,), jnp.int32),
               pltpu.VMEM((b_per_w, D), jnp.float32)]
    scratch += [pltpu.VMEM((CH,), jnp.int32) for _ in range(n_ch)]
    scratch += [pltpu.VMEM((CH, D), jnp.float32) for _ in range(n_ch)]
    scratch += [pltpu.VMEM((b_per_w, D), jnp.float32),
                pltpu.SemaphoreType.DMA]

    @functools.partial(
        pl.kernel,
        mesh=mesh,
        out_type=(jax.ShapeDtypeStruct((B, D), jnp.float32),
                  jax.ShapeDtypeStruct((B, D), jnp.float32)),
        scratch_types=scratch,
        compiler_params=pltpu.CompilerParams(use_tc_tiling_on_sc=False),
    )
    def sc_gather(cidx_hbm, ctx_hbm, wemb_hbm, wout_hbm, out_h, out_csum,
                  idx_h, rows_h, *rest):
        idx_c = rest[0:n_ch]
        rows_c = rest[n_ch:2 * n_ch]
        csum_v = rest[2 * n_ch]
        sem = rest[2 * n_ch + 1]

        wid = lax.axis_index("s") * NC + lax.axis_index("c")
        hbase = wid * b_per_w
        cbase = wid * c_per_w

        pltpu.sync_copy(cidx_hbm.at[pl.ds(hbase, b_per_w)], idx_h)
        for ch in range(n_ch):
            pltpu.sync_copy(ctx_hbm.at[pl.ds(cbase + ch * CH, CH)], idx_c[ch])

        cps = [pltpu.async_copy(wemb_hbm.at[idx_h], rows_h, sem)]
        for ch in range(n_ch):
            cps.append(pltpu.async_copy(wout_hbm.at[idx_c[ch]], rows_c[ch], sem))
        for cp in cps:
            cp.wait()

        for ch in range(n_ch):
            rc = rows_c[ch]

            def wbody(i, carry, rc=rc, off=ch * bpc):
                for l in range(nl):
                    acc = rc[i * Wn, pl.ds(l * 16, 16)]
                    for w in range(1, Wn):
                        acc = acc + rc[i * Wn + w, pl.ds(l * 16, 16)]
                    csum_v[off + i, pl.ds(l * 16, 16)] = acc
                return carry

            lax.fori_loop(0, bpc, wbody, 0)

        pltpu.sync_copy(rows_h, out_h.at[pl.ds(hbase, b_per_w)])
        pltpu.sync_copy(csum_v, out_csum.at[pl.ds(hbase, b_per_w)])

    return sc_gather


def _tc_body(h_ref, csum_ref, wout_ref, out_ref, m_ref, s_ref, hn_ref, *, B,
             Wn, V, tile_v):
    i = pl.program_id(0)
    nt = pl.num_programs(0)

    @pl.when(i == 0)
    def _init():
        m_ref[...] = jnp.full(m_ref.shape, _NEG, jnp.float32)
        s_ref[...] = jnp.zeros(s_ref.shape, jnp.float32)
        h = h_ref[...]
        hn_ref[...] = jnp.sqrt(jnp.sum(h * h, axis=1, keepdims=True)) * _LOG2E

    wt = wout_ref[...]
    rn2 = jnp.sum(wt * wt, axis=1, keepdims=True)
    mi = jnp.sqrt(jnp.max(rn2, axis=0, keepdims=True))
    b2w = (hn_ref[...] * mi).astype(jnp.bfloat16)
    b2 = b2w.astype(jnp.float32)

    hb = jnp.concatenate(
        [(h_ref[...] * _LOG2E).astype(jnp.bfloat16), b2w], axis=1)
    wb = jnp.concatenate(
        [wt.astype(jnp.bfloat16),
         jnp.full((tile_v, 1), -1.0, jnp.bfloat16)], axis=1)
    l2mb = lax.dot_general(hb, wb, (((1,), (1,)), ((), ())),
                           preferred_element_type=jnp.float32)
    t = jnp.sum(jnp.exp2(l2mb), axis=1, keepdims=True)

    m_old = m_ref[...]
    m_new = jnp.maximum(m_old, b2)
    s_ref[...] = (s_ref[...] * jnp.exp2(m_old - m_new)
                  + t * jnp.exp2(b2 - m_new))
    m_ref[...] = m_new

    @pl.when(i == nt - 1)
    def _fin():
        lse = m_ref[...] * _LN2 + jnp.log(s_ref[...])
        ctx_total = jnp.sum(h_ref[...] * csum_ref[...])
        out_ref[0, 0] = jnp.sum(lse) / B - ctx_total / (B * Wn)


@functools.lru_cache(maxsize=None)
def _make_tc_loss(B, Wn, V, D):
    tile_v = _TILE_V
    nt = pl.cdiv(V, tile_v)
    body = functools.partial(_tc_body, B=B, Wn=Wn, V=V, tile_v=tile_v)
    return pl.pallas_call(
        body,
        grid=(nt,),
        in_specs=[
            pl.BlockSpec((B, D), lambda i: (0, 0)),
            pl.BlockSpec((B, D), lambda i: (0, 0)),
            pl.BlockSpec((tile_v, D), lambda i: (i, 0)),
        ],
        out_specs=pl.BlockSpec((1, 1), lambda i: (0, 0),
                               memory_space=pltpu.SMEM),
        out_shape=jax.ShapeDtypeStruct((1, 1), jnp.float32),
        scratch_shapes=[
            pltpu.VMEM((B, 1), jnp.float32),
            pltpu.VMEM((B, 1), jnp.float32),
            pltpu.VMEM((B, 1), jnp.float32),
        ],
    )


def kernel(center_index, context_indices, W_emb, W_out):
    B, Wn = context_indices.shape
    V, D = W_emb.shape
    cidx = center_index.astype(jnp.int32)
    ctx = context_indices.astype(jnp.int32).reshape(-1)
    h, csum = _make_sc_gather(B, Wn, V, D)(cidx, ctx, W_emb, W_out)
    loss = _make_tc_loss(B, Wn, V, D)(h, csum, W_out)
    return loss[0, 0]


# speedup vs baseline: 2.1349x; 1.2635x over previous
import functools

import jax
import jax.numpy as jnp
from jax import lax
from jax.experimental import pallas as pl
from jax.experimental.pallas import tpu as pltpu
from jax.experimental.pallas import tpu_sc as plsc

_TILE_V = 2000
_NEG = -3e38
_LOG2E = 1.4426950408889634
_LN2 = 0.6931471805599453


@functools.lru_cache(maxsize=None)
def _make_sc_gather(B, Wn, V, D):
    info = plsc.get_sparse_core_info()
    NC, NS = info.num_cores, info.num_subcores
    NW = NC * NS
    assert B % NW == 0
    b_per_w = B // NW
    c_per_w = (B * Wn) // NW
    n_pairs = b_per_w // 2
    assert b_per_w % 2 == 0 and c_per_w == b_per_w * Wn
    assert D % 16 == 0
    nl = D // 16

    mesh = plsc.VectorSubcoreMesh(core_axis_name="c", subcore_axis_name="s")

    scratch = [
        pltpu.VMEM((b_per_w,), jnp.int32),
        pltpu.VMEM((c_per_w,), jnp.int32),
        pltpu.VMEM((b_per_w, D), jnp.float32),
        pltpu.VMEM((Wn, D), jnp.float32),
        pltpu.VMEM((Wn, D), jnp.float32),
        pltpu.VMEM((b_per_w, D), jnp.float32),
        pltpu.SemaphoreType.DMA,
        pltpu.SemaphoreType.DMA,
        pltpu.SemaphoreType.DMA,
    ]

    @functools.partial(
        pl.kernel,
        mesh=mesh,
        out_type=(jax.ShapeDtypeStruct((B, D), jnp.float32),
                  jax.ShapeDtypeStruct((B, D), jnp.float32)),
        scratch_types=scratch,
    )
    def sc_gather(cidx_hbm, ctx_hbm, wemb_hbm, wout_hbm, out_h, out_csum,
                  idx_h_s, idx_c_s, rows_h, buf_e, buf_o,
                  csum_v, sem_h, sem_e, sem_o):
        wid = lax.axis_index("s") * NC + lax.axis_index("c")
        hbase = wid * b_per_w
        cbase = wid * c_per_w

        pltpu.sync_copy(cidx_hbm.at[pl.ds(hbase, b_per_w)], idx_h_s)
        pltpu.sync_copy(ctx_hbm.at[pl.ds(cbase, c_per_w)], idx_c_s)

        for j0 in range(0, b_per_w, 16):
            hvec = idx_h_s[pl.ds(j0, 16)]
            for j in range(16):
                pltpu.async_copy(wemb_hbm.at[pl.ds(hvec[j], 1), :],
                                 rows_h.at[pl.ds(j0 + j, 1), :], sem_h)

        def fire(buf, sem, cvec, lane0):
            for j in range(Wn):
                pltpu.async_copy(wout_hbm.at[pl.ds(cvec[lane0 + j], 1), :],
                                 buf.at[pl.ds(j, 1), :], sem)

        def drain(buf, sem):
            pltpu.make_async_copy(wout_hbm.at[pl.ds(0, Wn), :], buf, sem).wait()

        def wsum(buf, chunk):
            for l in range(nl):
                acc = buf[0, pl.ds(l * 16, 16)]
                for w in range(1, Wn):
                    acc = acc + buf[w, pl.ds(l * 16, 16)]
                csum_v[chunk, pl.ds(l * 16, 16)] = acc

        def body(i, carry):
            cvec = idx_c_s[pl.ds(i * 2 * Wn, 16)]
            fire(buf_e, sem_e, cvec, 0)

            @pl.when(i > 0)
            def _():
                drain(buf_o, sem_o)
                wsum(buf_o, 2 * i - 1)

            fire(buf_o, sem_o, cvec, Wn)
            drain(buf_e, sem_e)
            wsum(buf_e, 2 * i)
            return carry

        lax.fori_loop(0, n_pairs, body, 0)
        drain(buf_o, sem_o)
        wsum(buf_o, b_per_w - 1)

        pltpu.make_async_copy(wemb_hbm.at[pl.ds(0, b_per_w), :], rows_h,
                              sem_h).wait()
        pltpu.sync_copy(rows_h, out_h.at[pl.ds(hbase, b_per_w)])
        pltpu.sync_copy(csum_v, out_csum.at[pl.ds(hbase, b_per_w)])

    return sc_gather


def _tc_body(h_ref, csum_ref, wout_ref, out_ref, m_ref, s_ref, hn_ref, *, B,
             Wn, V, tile_v):
    i = pl.program_id(0)
    nt = pl.num_programs(0)

    @pl.when(i == 0)
    def _init():
        m_ref[...] = jnp.full(m_ref.shape, _NEG, jnp.float32)
        s_ref[...] = jnp.zeros(s_ref.shape, jnp.float32)
        h = h_ref[...]
        hn_ref[...] = jnp.sqrt(jnp.sum(h * h, axis=1, keepdims=True)) * _LOG2E

    wt = wout_ref[...]
    rn2 = jnp.sum(wt * wt, axis=1, keepdims=True)
    mi = jnp.sqrt(jnp.max(rn2, axis=0, keepdims=True))
    b2w = (hn_ref[...] * mi).astype(jnp.bfloat16)
    b2 = b2w.astype(jnp.float32)

    hb = jnp.concatenate(
        [(h_ref[...] * _LOG2E).astype(jnp.bfloat16), b2w], axis=1)
    wb = jnp.concatenate(
        [wt.astype(jnp.bfloat16),
         jnp.full((tile_v, 1), -1.0, jnp.bfloat16)], axis=1)
    l2mb = lax.dot_general(hb, wb, (((1,), (1,)), ((), ())),
                           preferred_element_type=jnp.float32)
    t = jnp.sum(jnp.exp2(l2mb), axis=1, keepdims=True)

    m_old = m_ref[...]
    m_new = jnp.maximum(m_old, b2)
    s_ref[...] = (s_ref[...] * jnp.exp2(m_old - m_new)
                  + t * jnp.exp2(b2 - m_new))
    m_ref[...] = m_new

    @pl.when(i == nt - 1)
    def _fin():
        lse = m_ref[...] * _LN2 + jnp.log(s_ref[...])
        ctx_total = jnp.sum(h_ref[...] * csum_ref[...])
        out_ref[0, 0] = jnp.sum(lse) / B - ctx_total / (B * Wn)


@functools.lru_cache(maxsize=None)
def _make_tc_loss(B, Wn, V, D):
    tile_v = _TILE_V
    nt = pl.cdiv(V, tile_v)
    body = functools.partial(_tc_body, B=B, Wn=Wn, V=V, tile_v=tile_v)
    return pl.pallas_call(
        body,
        grid=(nt,),
        in_specs=[
            pl.BlockSpec((B, D), lambda i: (0, 0)),
            pl.BlockSpec((B, D), lambda i: (0, 0)),
            pl.BlockSpec((tile_v, D), lambda i: (i, 0)),
        ],
        out_specs=pl.BlockSpec((1, 1), lambda i: (0, 0),
                               memory_space=pltpu.SMEM),
        out_shape=jax.ShapeDtypeStruct((1, 1), jnp.float32),
        scratch_shapes=[
            pltpu.VMEM((B, 1), jnp.float32),
            pltpu.VMEM((B, 1), jnp.float32),
            pltpu.VMEM((B, 1), jnp.float32),
        ],
    )


def kernel(center_index, context_indices, W_emb, W_out):
    B, Wn = context_indices.shape
    V, D = W_emb.shape
    cidx = center_index.astype(jnp.int32)
    ctx = context_indices.astype(jnp.int32).reshape(-1)
    h, csum = _make_sc_gather(B, Wn, V, D)(cidx, ctx, W_emb, W_out)
    loss = _make_tc_loss(B, Wn, V, D)(h, csum, W_out)
    return loss[0, 0]


# speedup vs baseline: 2.2812x; 1.0685x over previous
import functools

import jax
import jax.numpy as jnp
from jax import lax
from jax.experimental import pallas as pl
from jax.experimental.pallas import tpu as pltpu
from jax.experimental.pallas import tpu_sc as plsc

_TILE_V = 4000
_NEG = -3e38
_LOG2E = 1.4426950408889634
_LN2 = 0.6931471805599453


@functools.lru_cache(maxsize=None)
def _make_sc_gather(B, Wn, V, D):
    info = plsc.get_sparse_core_info()
    NC, NS = info.num_cores, info.num_subcores
    NW = NC * NS
    assert B % NW == 0
    b_per_w = B // NW
    c_per_w = (B * Wn) // NW
    n_pairs = b_per_w // 2
    assert b_per_w % 2 == 0 and c_per_w == b_per_w * Wn
    assert D % 16 == 0
    nl = D // 16

    mesh = plsc.VectorSubcoreMesh(core_axis_name="c", subcore_axis_name="s")

    scratch = [
        pltpu.VMEM((b_per_w,), jnp.int32),
        pltpu.VMEM((c_per_w,), jnp.int32),
        pltpu.VMEM((b_per_w, D), jnp.float32),
        pltpu.VMEM((Wn, D), jnp.float32),
        pltpu.VMEM((Wn, D), jnp.float32),
        pltpu.VMEM((b_per_w, D), jnp.float32),
        pltpu.SemaphoreType.DMA,
        pltpu.SemaphoreType.DMA,
        pltpu.SemaphoreType.DMA,
    ]

    @functools.partial(
        pl.kernel,
        mesh=mesh,
        out_type=(jax.ShapeDtypeStruct((B, D), jnp.float32),
                  jax.ShapeDtypeStruct((B, D), jnp.float32)),
        scratch_types=scratch,
    )
    def sc_gather(cidx_hbm, ctx_hbm, wemb_hbm, wout_hbm, out_h, out_csum,
                  idx_h_s, idx_c_s, rows_h, buf_e, buf_o,
                  csum_v, sem_h, sem_e, sem_o):
        wid = lax.axis_index("s") * NC + lax.axis_index("c")
        hbase = wid * b_per_w
        cbase = wid * c_per_w

        pltpu.sync_copy(cidx_hbm.at[pl.ds(hbase, b_per_w)], idx_h_s)
        pltpu.sync_copy(ctx_hbm.at[pl.ds(cbase, c_per_w)], idx_c_s)

        for j0 in range(0, b_per_w, 16):
            hvec = idx_h_s[pl.ds(j0, 16)]
            for j in range(16):
                pltpu.async_copy(wemb_hbm.at[pl.ds(hvec[j], 1), :],
                                 rows_h.at[pl.ds(j0 + j, 1), :], sem_h)

        def fire(buf, sem, cvec, lane0):
            for j in range(Wn):
                pltpu.async_copy(wout_hbm.at[pl.ds(cvec[lane0 + j], 1), :],
                                 buf.at[pl.ds(j, 1), :], sem)

        def drain(buf, sem):
            pltpu.make_async_copy(wout_hbm.at[pl.ds(0, Wn), :], buf, sem).wait()

        def wsum(buf, chunk):
            for l in range(nl):
                acc = buf[0, pl.ds(l * 16, 16)]
                for w in range(1, Wn):
                    acc = acc + buf[w, pl.ds(l * 16, 16)]
                csum_v[chunk, pl.ds(l * 16, 16)] = acc

        def body(i, carry):
            cvec = idx_c_s[pl.ds(i * 2 * Wn, 16)]
            fire(buf_e, sem_e, cvec, 0)

            @pl.when(i > 0)
            def _():
                drain(buf_o, sem_o)
                wsum(buf_o, 2 * i - 1)

            fire(buf_o, sem_o, cvec, Wn)
            drain(buf_e, sem_e)
            wsum(buf_e, 2 * i)
            return carry

        lax.fori_loop(0, n_pairs, body, 0)
        drain(buf_o, sem_o)
        wsum(buf_o, b_per_w - 1)

        pltpu.make_async_copy(wemb_hbm.at[pl.ds(0, b_per_w), :], rows_h,
                              sem_h).wait()
        pltpu.sync_copy(rows_h, out_h.at[pl.ds(hbase, b_per_w)])
        pltpu.sync_copy(csum_v, out_csum.at[pl.ds(hbase, b_per_w)])

    return sc_gather


def _tc_body(h_ref, csum_ref, wout_ref, out_ref, m_ref, s_ref, hn_ref, *, B,
             Wn, V, tile_v):
    i = pl.program_id(0)
    nt = pl.num_programs(0)

    @pl.when(i == 0)
    def _init():
        m_ref[...] = jnp.full(m_ref.shape, _NEG, jnp.float32)
        s_ref[...] = jnp.zeros(s_ref.shape, jnp.float32)
        h = h_ref[...]
        hn_ref[...] = jnp.sqrt(jnp.sum(h * h, axis=1, keepdims=True)) * _LOG2E

    wt = wout_ref[...]
    rn2 = jnp.sum(wt * wt, axis=1, keepdims=True)
    mi = jnp.sqrt(jnp.max(rn2, axis=0, keepdims=True))
    b2w = (hn_ref[...] * mi).astype(jnp.bfloat16)
    b2 = b2w.astype(jnp.float32)

    hb = jnp.concatenate(
        [(h_ref[...] * _LOG2E).astype(jnp.bfloat16), b2w], axis=1)
    wb = jnp.concatenate(
        [wt.astype(jnp.bfloat16),
         jnp.full((tile_v, 1), -1.0, jnp.bfloat16)], axis=1)
    l2mb = lax.dot_general(hb, wb, (((1,), (1,)), ((), ())),
                           preferred_element_type=jnp.float32)
    t = jnp.sum(jnp.exp2(l2mb), axis=1, keepdims=True)

    m_old = m_ref[...]
    m_new = jnp.maximum(m_old, b2)
    s_ref[...] = (s_ref[...] * jnp.exp2(m_old - m_new)
                  + t * jnp.exp2(b2 - m_new))
    m_ref[...] = m_new

    @pl.when(i == nt - 1)
    def _fin():
        lse = m_ref[...] * _LN2 + jnp.log(s_ref[...])
        ctx_total = jnp.sum(h_ref[...] * csum_ref[...])
        out_ref[0, 0] = jnp.sum(lse) / B - ctx_total / (B * Wn)


@functools.lru_cache(maxsize=None)
def _make_tc_loss(B, Wn, V, D):
    tile_v = _TILE_V
    nt = pl.cdiv(V, tile_v)
    body = functools.partial(_tc_body, B=B, Wn=Wn, V=V, tile_v=tile_v)
    return pl.pallas_call(
        body,
        grid=(nt,),
        in_specs=[
            pl.BlockSpec((B, D), lambda i: (0, 0)),
            pl.BlockSpec((B, D), lambda i: (0, 0)),
            pl.BlockSpec((tile_v, D), lambda i: (i, 0)),
        ],
        out_specs=pl.BlockSpec((1, 1), lambda i: (0, 0),
                               memory_space=pltpu.SMEM),
        out_shape=jax.ShapeDtypeStruct((1, 1), jnp.float32),
        scratch_shapes=[
            pltpu.VMEM((B, 1), jnp.float32),
            pltpu.VMEM((B, 1), jnp.float32),
            pltpu.VMEM((B, 1), jnp.float32),
        ],
    )


def kernel(center_index, context_indices, W_emb, W_out):
    B, Wn = context_indices.shape
    V, D = W_emb.shape
    cidx = center_index.astype(jnp.int32)
    ctx = context_indices.astype(jnp.int32).reshape(-1)
    h, csum = _make_sc_gather(B, Wn, V, D)(cidx, ctx, W_emb, W_out)
    loss = _make_tc_loss(B, Wn, V, D)(h, csum, W_out)
    return loss[0, 0]


# speedup vs baseline: 2.3132x; 1.0140x over previous
import functools

import jax
import jax.numpy as jnp
from jax import lax
from jax.experimental import pallas as pl
from jax.experimental.pallas import tpu as pltpu
from jax.experimental.pallas import tpu_sc as plsc

_TILE_V = 5000
_NEG = -3e38
_LOG2E = 1.4426950408889634
_LN2 = 0.6931471805599453


@functools.lru_cache(maxsize=None)
def _make_sc_gather(B, Wn, V, D):
    info = plsc.get_sparse_core_info()
    NC, NS = info.num_cores, info.num_subcores
    NW = NC * NS
    assert B % NW == 0
    b_per_w = B // NW
    c_per_w = (B * Wn) // NW
    n_pairs = b_per_w // 2
    assert b_per_w % 2 == 0 and c_per_w == b_per_w * Wn
    assert D % 16 == 0
    nl = D // 16

    mesh = plsc.VectorSubcoreMesh(core_axis_name="c", subcore_axis_name="s")

    scratch = [
        pltpu.VMEM((b_per_w,), jnp.int32),
        pltpu.VMEM((c_per_w,), jnp.int32),
        pltpu.VMEM((b_per_w, D), jnp.float32),
        pltpu.VMEM((Wn, D), jnp.float32),
        pltpu.VMEM((Wn, D), jnp.float32),
        pltpu.VMEM((b_per_w, D), jnp.float32),
        pltpu.SemaphoreType.DMA,
        pltpu.SemaphoreType.DMA,
        pltpu.SemaphoreType.DMA,
    ]

    @functools.partial(
        pl.kernel,
        mesh=mesh,
        out_type=(jax.ShapeDtypeStruct((B, D), jnp.float32),
                  jax.ShapeDtypeStruct((B, D), jnp.float32)),
        scratch_types=scratch,
    )
    def sc_gather(cidx_hbm, ctx_hbm, wemb_hbm, wout_hbm, out_h, out_csum,
                  idx_h_s, idx_c_s, rows_h, buf_e, buf_o,
                  csum_v, sem_h, sem_e, sem_o):
        wid = lax.axis_index("s") * NC + lax.axis_index("c")
        hbase = wid * b_per_w
        cbase = wid * c_per_w

        pltpu.sync_copy(cidx_hbm.at[pl.ds(hbase, b_per_w)], idx_h_s)
        pltpu.sync_copy(ctx_hbm.at[pl.ds(cbase, c_per_w)], idx_c_s)

        for j0 in range(0, b_per_w, 16):
            hvec = idx_h_s[pl.ds(j0, 16)]
            for j in range(16):
                pltpu.async_copy(wemb_hbm.at[pl.ds(hvec[j], 1), :],
                                 rows_h.at[pl.ds(j0 + j, 1), :], sem_h)

        def fire(buf, sem, cvec, lane0):
            for j in range(Wn):
                pltpu.async_copy(wout_hbm.at[pl.ds(cvec[lane0 + j], 1), :],
                                 buf.at[pl.ds(j, 1), :], sem)

        def drain(buf, sem):
            pltpu.make_async_copy(wout_hbm.at[pl.ds(0, Wn), :], buf, sem).wait()

        def wsum(buf, chunk):
            for l in range(nl):
                acc = buf[0, pl.ds(l * 16, 16)]
                for w in range(1, Wn):
                    acc = acc + buf[w, pl.ds(l * 16, 16)]
                csum_v[chunk, pl.ds(l * 16, 16)] = acc

        def body(i, carry):
            cvec = idx_c_s[pl.ds(i * 2 * Wn, 16)]
            fire(buf_e, sem_e, cvec, 0)

            @pl.when(i > 0)
            def _():
                drain(buf_o, sem_o)
                wsum(buf_o, 2 * i - 1)

            fire(buf_o, sem_o, cvec, Wn)
            drain(buf_e, sem_e)
            wsum(buf_e, 2 * i)
            return carry

        lax.fori_loop(0, n_pairs, body, 0)
        drain(buf_o, sem_o)
        wsum(buf_o, b_per_w - 1)

        pltpu.make_async_copy(wemb_hbm.at[pl.ds(0, b_per_w), :], rows_h,
                              sem_h).wait()
        pltpu.sync_copy(rows_h, out_h.at[pl.ds(hbase, b_per_w)])
        pltpu.sync_copy(csum_v, out_csum.at[pl.ds(hbase, b_per_w)])

    return sc_gather


def _tc_body(h_ref, csum_ref, wout_ref, out_ref, m_ref, s_ref, hn_ref, *, B,
             Wn, V, tile_v):
    i = pl.program_id(0)
    nt = pl.num_programs(0)

    @pl.when(i == 0)
    def _init():
        m_ref[...] = jnp.full(m_ref.shape, _NEG, jnp.float32)
        s_ref[...] = jnp.zeros(s_ref.shape, jnp.float32)
        h = h_ref[...]
        hn_ref[...] = jnp.sqrt(jnp.sum(h * h, axis=1, keepdims=True)) * _LOG2E

    wt = wout_ref[...]
    rn2 = jnp.sum(wt * wt, axis=1, keepdims=True)
    mi = jnp.sqrt(jnp.max(rn2, axis=0, keepdims=True))
    b2w = (hn_ref[...] * mi).astype(jnp.bfloat16)
    b2 = b2w.astype(jnp.float32)

    hb = jnp.concatenate(
        [(h_ref[...] * _LOG2E).astype(jnp.bfloat16), b2w], axis=1)
    wb = jnp.concatenate(
        [wt.astype(jnp.bfloat16),
         jnp.full((tile_v, 1), -1.0, jnp.bfloat16)], axis=1)
    l2mb = lax.dot_general(hb, wb, (((1,), (1,)), ((), ())),
                           preferred_element_type=jnp.float32)
    t = jnp.sum(jnp.exp2(l2mb), axis=1, keepdims=True)

    m_old = m_ref[...]
    m_new = jnp.maximum(m_old, b2)
    s_ref[...] = (s_ref[...] * jnp.exp2(m_old - m_new)
                  + t * jnp.exp2(b2 - m_new))
    m_ref[...] = m_new

    @pl.when(i == nt - 1)
    def _fin():
        lse = m_ref[...] * _LN2 + jnp.log(s_ref[...])
        ctx_total = jnp.sum(h_ref[...] * csum_ref[...])
        out_ref[0, 0] = jnp.sum(lse) / B - ctx_total / (B * Wn)


@functools.lru_cache(maxsize=None)
def _make_tc_loss(B, Wn, V, D):
    tile_v = _TILE_V
    nt = pl.cdiv(V, tile_v)
    body = functools.partial(_tc_body, B=B, Wn=Wn, V=V, tile_v=tile_v)
    return pl.pallas_call(
        body,
        grid=(nt,),
        in_specs=[
            pl.BlockSpec((B, D), lambda i: (0, 0)),
            pl.BlockSpec((B, D), lambda i: (0, 0)),
            pl.BlockSpec((tile_v, D), lambda i: (i, 0)),
        ],
        out_specs=pl.BlockSpec((1, 1), lambda i: (0, 0),
                               memory_space=pltpu.SMEM),
        out_shape=jax.ShapeDtypeStruct((1, 1), jnp.float32),
        scratch_shapes=[
            pltpu.VMEM((B, 1), jnp.float32),
            pltpu.VMEM((B, 1), jnp.float32),
            pltpu.VMEM((B, 1), jnp.float32),
        ],
    )


def kernel(center_index, context_indices, W_emb, W_out):
    B, Wn = context_indices.shape
    V, D = W_emb.shape
    cidx = center_index.astype(jnp.int32)
    ctx = context_indices.astype(jnp.int32).reshape(-1)
    h, csum = _make_sc_gather(B, Wn, V, D)(cidx, ctx, W_emb, W_out)
    loss = _make_tc_loss(B, Wn, V, D)(h, csum, W_out)
    return loss[0, 0]


# speedup vs baseline: 2.3843x; 1.0308x over previous
import functools

import jax
import jax.numpy as jnp
from jax import lax
from jax.experimental import pallas as pl
from jax.experimental.pallas import tpu as pltpu
from jax.experimental.pallas import tpu_sc as plsc

_TILE_V = 10000
_NEG = -3e38
_LOG2E = 1.4426950408889634
_LN2 = 0.6931471805599453


@functools.lru_cache(maxsize=None)
def _make_sc_gather(B, Wn, V, D):
    info = plsc.get_sparse_core_info()
    NC, NS = info.num_cores, info.num_subcores
    NW = NC * NS
    assert B % NW == 0
    b_per_w = B // NW
    c_per_w = (B * Wn) // NW
    n_pairs = b_per_w // 2
    assert b_per_w % 2 == 0 and c_per_w == b_per_w * Wn
    assert D % 16 == 0
    nl = D // 16

    mesh = plsc.VectorSubcoreMesh(core_axis_name="c", subcore_axis_name="s")

    scratch = [
        pltpu.VMEM((b_per_w,), jnp.int32),
        pltpu.VMEM((c_per_w,), jnp.int32),
        pltpu.VMEM((b_per_w, D), jnp.float32),
        pltpu.VMEM((Wn, D), jnp.float32),
        pltpu.VMEM((Wn, D), jnp.float32),
        pltpu.VMEM((b_per_w, D), jnp.float32),
        pltpu.SemaphoreType.DMA,
        pltpu.SemaphoreType.DMA,
        pltpu.SemaphoreType.DMA,
    ]

    @functools.partial(
        pl.kernel,
        mesh=mesh,
        out_type=(jax.ShapeDtypeStruct((B, D), jnp.float32),
                  jax.ShapeDtypeStruct((B, D), jnp.float32)),
        scratch_types=scratch,
    )
    def sc_gather(cidx_hbm, ctx_hbm, wemb_hbm, wout_hbm, out_h, out_csum,
                  idx_h_s, idx_c_s, rows_h, buf_e, buf_o,
                  csum_v, sem_h, sem_e, sem_o):
        wid = lax.axis_index("s") * NC + lax.axis_index("c")
        hbase = wid * b_per_w
        cbase = wid * c_per_w

        pltpu.sync_copy(cidx_hbm.at[pl.ds(hbase, b_per_w)], idx_h_s)
        pltpu.sync_copy(ctx_hbm.at[pl.ds(cbase, c_per_w)], idx_c_s)

        for j0 in range(0, b_per_w, 16):
            hvec = idx_h_s[pl.ds(j0, 16)]
            for j in range(16):
                pltpu.async_copy(wemb_hbm.at[pl.ds(hvec[j], 1), :],
                                 rows_h.at[pl.ds(j0 + j, 1), :], sem_h)

        def fire(buf, sem, cvec, lane0):
            for j in range(Wn):
                pltpu.async_copy(wout_hbm.at[pl.ds(cvec[lane0 + j], 1), :],
                                 buf.at[pl.ds(j, 1), :], sem)

        def drain(buf, sem):
            pltpu.make_async_copy(wout_hbm.at[pl.ds(0, Wn), :], buf, sem).wait()

        def wsum(buf, chunk):
            for l in range(nl):
                acc = buf[0, pl.ds(l * 16, 16)]
                for w in range(1, Wn):
                    acc = acc + buf[w, pl.ds(l * 16, 16)]
                csum_v[chunk, pl.ds(l * 16, 16)] = acc

        def body(i, carry):
            cvec = idx_c_s[pl.ds(i * 2 * Wn, 16)]
            fire(buf_e, sem_e, cvec, 0)

            @pl.when(i > 0)
            def _():
                drain(buf_o, sem_o)
                wsum(buf_o, 2 * i - 1)

            fire(buf_o, sem_o, cvec, Wn)
            drain(buf_e, sem_e)
            wsum(buf_e, 2 * i)
            return carry

        lax.fori_loop(0, n_pairs, body, 0)
        drain(buf_o, sem_o)
        wsum(buf_o, b_per_w - 1)

        pltpu.make_async_copy(wemb_hbm.at[pl.ds(0, b_per_w), :], rows_h,
                              sem_h).wait()
        pltpu.sync_copy(rows_h, out_h.at[pl.ds(hbase, b_per_w)])
        pltpu.sync_copy(csum_v, out_csum.at[pl.ds(hbase, b_per_w)])

    return sc_gather


def _tc_body(h_ref, csum_ref, wout_ref, out_ref, m_ref, s_ref, hn_ref, *, B,
             Wn, V, tile_v):
    i = pl.program_id(0)
    nt = pl.num_programs(0)

    @pl.when(i == 0)
    def _init():
        m_ref[...] = jnp.full(m_ref.shape, _NEG, jnp.float32)
        s_ref[...] = jnp.zeros(s_ref.shape, jnp.float32)
        h = h_ref[...]
        hn_ref[...] = jnp.sqrt(jnp.sum(h * h, axis=1, keepdims=True)) * _LOG2E

    wt = wout_ref[...]
    rn2 = jnp.sum(wt * wt, axis=1, keepdims=True)
    mi = jnp.sqrt(jnp.max(rn2, axis=0, keepdims=True))
    b2w = (hn_ref[...] * mi).astype(jnp.bfloat16)
    b2 = b2w.astype(jnp.float32)

    hb = jnp.concatenate(
        [(h_ref[...] * _LOG2E).astype(jnp.bfloat16), b2w], axis=1)
    wb = jnp.concatenate(
        [wt.astype(jnp.bfloat16),
         jnp.full((tile_v, 1), -1.0, jnp.bfloat16)], axis=1)
    l2mb = lax.dot_general(hb, wb, (((1,), (1,)), ((), ())),
                           preferred_element_type=jnp.float32)
    t = jnp.sum(jnp.exp2(l2mb), axis=1, keepdims=True)

    m_old = m_ref[...]
    m_new = jnp.maximum(m_old, b2)
    s_ref[...] = (s_ref[...] * jnp.exp2(m_old - m_new)
                  + t * jnp.exp2(b2 - m_new))
    m_ref[...] = m_new

    @pl.when(i == nt - 1)
    def _fin():
        lse = m_ref[...] * _LN2 + jnp.log(s_ref[...])
        ctx_total = jnp.sum(h_ref[...] * csum_ref[...])
        out_ref[0, 0] = jnp.sum(lse) / B - ctx_total / (B * Wn)


@functools.lru_cache(maxsize=None)
def _make_tc_loss(B, Wn, V, D):
    tile_v = _TILE_V
    nt = pl.cdiv(V, tile_v)
    body = functools.partial(_tc_body, B=B, Wn=Wn, V=V, tile_v=tile_v)
    return pl.pallas_call(
        body,
        grid=(nt,),
        in_specs=[
            pl.BlockSpec((B, D), lambda i: (0, 0)),
            pl.BlockSpec((B, D), lambda i: (0, 0)),
            pl.BlockSpec((tile_v, D), lambda i: (i, 0)),
        ],
        out_specs=pl.BlockSpec((1, 1), lambda i: (0, 0),
                               memory_space=pltpu.SMEM),
        out_shape=jax.ShapeDtypeStruct((1, 1), jnp.float32),
        scratch_shapes=[
            pltpu.VMEM((B, 1), jnp.float32),
            pltpu.VMEM((B, 1), jnp.float32),
            pltpu.VMEM((B, 1), jnp.float32),
        ],
    )


def kernel(center_index, context_indices, W_emb, W_out):
    B, Wn = context_indices.shape
    V, D = W_emb.shape
    cidx = center_index.astype(jnp.int32)
    ctx = context_indices.astype(jnp.int32).reshape(-1)
    h, csum = _make_sc_gather(B, Wn, V, D)(cidx, ctx, W_emb, W_out)
    loss = _make_tc_loss(B, Wn, V, D)(h, csum, W_out)
    return loss[0, 0]


# speedup vs baseline: 2.3853x; 1.0004x over previous
import functools

import jax
import jax.numpy as jnp
from jax import lax
from jax.experimental import pallas as pl
from jax.experimental.pallas import tpu as pltpu
from jax.experimental.pallas import tpu_sc as plsc

_TILE_V = 10000
_NEG = -3e38
_LOG2E = 1.4426950408889634
_LN2 = 0.6931471805599453


@functools.lru_cache(maxsize=None)
def _make_sc_gather(B, Wn, V, D):
    info = plsc.get_sparse_core_info()
    NC, NS = info.num_cores, info.num_subcores
    NW = NC * NS
    assert B % NW == 0
    b_per_w = B // NW
    c_per_w = (B * Wn) // NW
    n_pairs = b_per_w // 2
    assert b_per_w % 2 == 0 and c_per_w == b_per_w * Wn
    assert D % 16 == 0
    nl = D // 16

    mesh = plsc.VectorSubcoreMesh(core_axis_name="c", subcore_axis_name="s")

    scratch = [
        pltpu.VMEM((b_per_w,), jnp.int32),
        pltpu.VMEM((c_per_w,), jnp.int32),
        pltpu.VMEM((b_per_w, D), jnp.float32),
        pltpu.VMEM((Wn, D), jnp.float32),
        pltpu.VMEM((Wn, D), jnp.float32),
        pltpu.VMEM((b_per_w, D), jnp.float32),
        pltpu.SemaphoreType.DMA,
        pltpu.SemaphoreType.DMA,
        pltpu.SemaphoreType.DMA,
    ]

    @functools.partial(
        pl.kernel,
        mesh=mesh,
        out_type=(jax.ShapeDtypeStruct((B, D), jnp.float32),
                  jax.ShapeDtypeStruct((B, D), jnp.float32)),
        scratch_types=scratch,
        compiler_params=pltpu.CompilerParams(use_tc_tiling_on_sc=True),
    )
    def sc_gather(cidx_hbm, ctx_hbm, wemb_hbm, wout_hbm, out_h, out_csum,
                  idx_h_s, idx_c_s, rows_h, buf_e, buf_o,
                  csum_v, sem_h, sem_e, sem_o):
        wid = lax.axis_index("s") * NC + lax.axis_index("c")
        hbase = wid * b_per_w
        cbase = wid * c_per_w

        pltpu.sync_copy(cidx_hbm.at[pl.ds(hbase, b_per_w)], idx_h_s)
        pltpu.sync_copy(ctx_hbm.at[pl.ds(cbase, c_per_w)], idx_c_s)

        for j0 in range(0, b_per_w, 16):
            hvec = idx_h_s[pl.ds(j0, 16)]
            for j in range(16):
                pltpu.async_copy(wemb_hbm.at[pl.ds(hvec[j], 1), :],
                                 rows_h.at[pl.ds(j0 + j, 1), :], sem_h)

        def fire(buf, sem, cvec, lane0):
            for j in range(Wn):
                pltpu.async_copy(wout_hbm.at[pl.ds(cvec[lane0 + j], 1), :],
                                 buf.at[pl.ds(j, 1), :], sem)

        def drain(buf, sem):
            pltpu.make_async_copy(wout_hbm.at[pl.ds(0, Wn), :], buf, sem).wait()

        def wsum(buf, chunk):
            for l in range(nl):
                acc = buf[0, pl.ds(l * 16, 16)]
                for w in range(1, Wn):
                    acc = acc + buf[w, pl.ds(l * 16, 16)]
                csum_v[chunk, pl.ds(l * 16, 16)] = acc

        def body(i, carry):
            cvec = idx_c_s[pl.ds(i * 2 * Wn, 16)]
            fire(buf_e, sem_e, cvec, 0)

            @pl.when(i > 0)
            def _():
                drain(buf_o, sem_o)
                wsum(buf_o, 2 * i - 1)

            fire(buf_o, sem_o, cvec, Wn)
            drain(buf_e, sem_e)
            wsum(buf_e, 2 * i)
            return carry

        lax.fori_loop(0, n_pairs, body, 0)
        drain(buf_o, sem_o)
        wsum(buf_o, b_per_w - 1)

        pltpu.make_async_copy(wemb_hbm.at[pl.ds(0, b_per_w), :], rows_h,
                              sem_h).wait()
        pltpu.sync_copy(rows_h, out_h.at[pl.ds(hbase, b_per_w)])
        pltpu.sync_copy(csum_v, out_csum.at[pl.ds(hbase, b_per_w)])

    return sc_gather


def _tc_body(h_ref, csum_ref, wout_ref, out_ref, m_ref, s_ref, hn_ref, *, B,
             Wn, V, tile_v):
    i = pl.program_id(0)
    nt = pl.num_programs(0)

    @pl.when(i == 0)
    def _init():
        m_ref[...] = jnp.full(m_ref.shape, _NEG, jnp.float32)
        s_ref[...] = jnp.zeros(s_ref.shape, jnp.float32)
        h = h_ref[...]
        hn_ref[...] = jnp.sqrt(jnp.sum(h * h, axis=1, keepdims=True)) * _LOG2E

    wt = wout_ref[...]
    rn2 = jnp.sum(wt * wt, axis=1, keepdims=True)
    mi = jnp.sqrt(jnp.max(rn2, axis=0, keepdims=True))
    b2w = (hn_ref[...] * mi).astype(jnp.bfloat16)
    b2 = b2w.astype(jnp.float32)

    hb = jnp.concatenate(
        [(h_ref[...] * _LOG2E).astype(jnp.bfloat16), b2w], axis=1)
    wb = jnp.concatenate(
        [wt.astype(jnp.bfloat16),
         jnp.full((tile_v, 1), -1.0, jnp.bfloat16)], axis=1)
    l2mb = lax.dot_general(hb, wb, (((1,), (1,)), ((), ())),
                           preferred_element_type=jnp.float32)
    t = jnp.sum(jnp.exp2(l2mb), axis=1, keepdims=True)

    m_old = m_ref[...]
    m_new = jnp.maximum(m_old, b2)
    s_ref[...] = (s_ref[...] * jnp.exp2(m_old - m_new)
                  + t * jnp.exp2(b2 - m_new))
    m_ref[...] = m_new

    @pl.when(i == nt - 1)
    def _fin():
        lse = m_ref[...] * _LN2 + jnp.log(s_ref[...])
        ctx_total = jnp.sum(h_ref[...] * csum_ref[...])
        out_ref[0, 0] = jnp.sum(lse) / B - ctx_total / (B * Wn)


@functools.lru_cache(maxsize=None)
def _make_tc_loss(B, Wn, V, D):
    tile_v = _TILE_V
    nt = pl.cdiv(V, tile_v)
    body = functools.partial(_tc_body, B=B, Wn=Wn, V=V, tile_v=tile_v)
    return pl.pallas_call(
        body,
        grid=(nt,),
        in_specs=[
            pl.BlockSpec((B, D), lambda i: (0, 0)),
            pl.BlockSpec((B, D), lambda i: (0, 0)),
            pl.BlockSpec((tile_v, D), lambda i: (i, 0)),
        ],
        out_specs=pl.BlockSpec((1, 1), lambda i: (0, 0),
                               memory_space=pltpu.SMEM),
        out_shape=jax.ShapeDtypeStruct((1, 1), jnp.float32),
        scratch_shapes=[
            pltpu.VMEM((B, 1), jnp.float32),
            pltpu.VMEM((B, 1), jnp.float32),
            pltpu.VMEM((B, 1), jnp.float32),
        ],
    )


def kernel(center_index, context_indices, W_emb, W_out):
    B, Wn = context_indices.shape
    V, D = W_emb.shape
    cidx = center_index.astype(jnp.int32)
    ctx = context_indices.astype(jnp.int32).reshape(-1)
    h, csum = _make_sc_gather(B, Wn, V, D)(cidx, ctx, W_emb, W_out)
    loss = _make_tc_loss(B, Wn, V, D)(h, csum, W_out)
    return loss[0, 0]
